# Initial kernel scaffold; baseline (speedup 1.0000x reference)
import functools

import jax
import jax.numpy as jnp
from jax import lax
from jax.experimental import pallas as pl
from jax.experimental.pallas import tpu as pltpu
from jax.experimental.pallas import tpu_sc as plsc

NNZ = 204800
BATCH = 4096
DIM = 64
NC = 2
NS = 16
NW = NC * NS
CHUNK = NNZ // NW
K = 128
NSUB = CHUNK // K
WCOL = 80
ROWS_PER_TILE = BATCH // NS


def _sc_body(ids_hbm, segs_hbm, w_hbm, table_hbm, out_hbm,
             ids_v, segs_v, w_v, rows_v, sbuf, bounce, accum, sem):
    cid = lax.axis_index("c")
    sid = lax.axis_index("s")
    wid = cid * NS + sid

    pltpu.sync_copy(ids_hbm.at[wid], ids_v)
    pltpu.sync_copy(segs_hbm.at[wid], segs_v)
    pltpu.sync_copy(w_hbm.at[wid], w_v)

    zeros = jnp.zeros((16,), jnp.float32)

    @pl.loop(0, K)
    def _zero_row(r):
        for c in range(WCOL // 16):
            sbuf[r, pl.ds(c * 16, 16)] = zeros

    pltpu.sync_copy(sbuf, accum.at[pl.ds(sid * ROWS_PER_TILE, K)])
    pltpu.sync_copy(sbuf, accum.at[pl.ds(sid * ROWS_PER_TILE + K, K)])
    plsc.subcore_barrier()

    lane = lax.iota(jnp.int32, 16)
    col64 = jnp.full((16,), DIM, jnp.int32)

    @pl.loop(0, NSUB)
    def _subchunk(g):
        idx = ids_v.at[pl.ds(g * K, K)]
        pltpu.async_copy(table_hbm.at[idx], rows_v, sem).wait()

        @pl.loop(0, K)
        def _row(r):
            w = w_v[g * K + r]
            for c in range(DIM // 16):
                sbuf[r, pl.ds(c * 16, 16)] = rows_v[r, pl.ds(c * 16, 16)] * w

        @pl.loop(0, K // 16)
        def _wcol(rr):
            w16 = w_v[pl.ds(g * K + rr * 16, 16)]
            plsc.store_scatter(sbuf, [rr * 16 + lane, col64], w16)

        pltpu.sync_copy(sbuf, accum.at[segs_v.at[g]], add=True)

    plsc.subcore_barrier()

    pltpu.sync_copy(accum.at[pl.ds(sid * ROWS_PER_TILE, ROWS_PER_TILE)], bounce)
    pltpu.sync_copy(bounce, out_hbm.at[cid, pl.ds(sid * ROWS_PER_TILE, ROWS_PER_TILE)])


_sc_lookup = functools.partial(
    pl.kernel,
    out_type=jax.ShapeDtypeStruct((NC, BATCH, WCOL), jnp.float32),
    mesh=plsc.VectorSubcoreMesh(
        core_axis_name="c", subcore_axis_name="s", num_cores=NC, num_subcores=NS
    ),
    scratch_types=[
        pltpu.VMEM((CHUNK,), jnp.int32),
        pltpu.VMEM((NSUB, 1, K), jnp.int32),
        pltpu.VMEM((CHUNK,), jnp.float32),
        pltpu.VMEM((K, DIM), jnp.float32),
        pltpu.VMEM((K, WCOL), jnp.float32),
        pltpu.VMEM((ROWS_PER_TILE, WCOL), jnp.float32),
        pltpu.VMEM_SHARED((BATCH, WCOL), jnp.float32),
        pltpu.SemaphoreType.DMA,
    ],
)(_sc_body)


def _combine_body(p_ref, o_ref):
    s = p_ref[0] + p_ref[1]
    denom = jnp.maximum(s[:, DIM:DIM + 1], 1e-12)
    o_ref[...] = s[:, :DIM] / denom


_combine = pl.pallas_call(
    _combine_body,
    out_shape=jax.ShapeDtypeStruct((BATCH, DIM), jnp.float32),
)


@jax.jit
def kernel(sp_ids_values, sp_ids_segment_ids, sp_weights_values, table):
    ids = sp_ids_values.reshape(NW, CHUNK)
    segs = sp_ids_segment_ids.reshape(NW, NSUB, 1, K)
    w = sp_weights_values.reshape(NW, CHUNK)
    partials = _sc_lookup(ids, segs, w, table)
    return _combine(partials)


# speedup vs baseline: 5.6402x; 5.6402x over previous
import functools

import jax
import jax.numpy as jnp
from jax import lax
from jax.experimental import pallas as pl
from jax.experimental.pallas import tpu as pltpu
from jax.experimental.pallas import tpu_sc as plsc

NNZ = 204800
BATCH = 4096
DIM = 64
NC = 2
NS = 16
NW = NC * NS
CHUNK = NNZ // NW
K = 128
NSUB = CHUNK // K
WCOL = 80
ROWS_PER_TILE = BATCH // NS


def _sc_body(ids_hbm, segs_hbm, w_hbm, table_hbm, out_hbm,
             ids_v, segs_v, w_v, rows_v, sbuf, bounce, accum, sem):
    cid = lax.axis_index("c")
    sid = lax.axis_index("s")
    wid = cid * NS + sid

    pltpu.sync_copy(ids_hbm.at[wid], ids_v)
    pltpu.sync_copy(segs_hbm.at[wid], segs_v)
    pltpu.sync_copy(w_hbm.at[wid], w_v)

    zeros = jnp.zeros((16,), jnp.float32)

    @pl.loop(0, K)
    def _zero_row(r):
        for c in range(WCOL // 16):
            sbuf[r, pl.ds(c * 16, 16)] = zeros

    pltpu.sync_copy(sbuf, accum.at[pl.ds(sid * ROWS_PER_TILE, K)])
    pltpu.sync_copy(sbuf, accum.at[pl.ds(sid * ROWS_PER_TILE + K, K)])
    plsc.subcore_barrier()

    @pl.loop(0, NSUB)
    def _subchunk(g):
        idx = ids_v.at[pl.ds(g * K, K)]
        pltpu.async_copy(table_hbm.at[idx], rows_v, sem).wait()

        @pl.loop(0, K // 16)
        def _rows16(rr):
            w16 = w_v[pl.ds(g * K + rr * 16, 16)]
            for r in range(16):
                w = w16[r]
                row = rr * 16 + r
                for c in range(DIM // 16):
                    sbuf[row, pl.ds(c * 16, 16)] = rows_v[row, pl.ds(c * 16, 16)] * w
                sbuf[row, pl.ds(DIM, 16)] = jnp.full((16,), w, jnp.float32)

        pltpu.sync_copy(sbuf, accum.at[segs_v.at[g]], add=True)

    plsc.subcore_barrier()

    pltpu.sync_copy(accum.at[pl.ds(sid * ROWS_PER_TILE, ROWS_PER_TILE)], bounce)
    pltpu.sync_copy(bounce, out_hbm.at[cid, pl.ds(sid * ROWS_PER_TILE, ROWS_PER_TILE)])


_sc_lookup = functools.partial(
    pl.kernel,
    out_type=jax.ShapeDtypeStruct((NC, BATCH, WCOL), jnp.float32),
    mesh=plsc.VectorSubcoreMesh(
        core_axis_name="c", subcore_axis_name="s", num_cores=NC, num_subcores=NS
    ),
    compiler_params=pltpu.CompilerParams(use_tc_tiling_on_sc=False),
    scratch_types=[
        pltpu.VMEM((CHUNK,), jnp.int32),
        pltpu.VMEM((NSUB, K), jnp.int32),
        pltpu.VMEM((CHUNK,), jnp.float32),
        pltpu.VMEM((K, DIM), jnp.float32),
        pltpu.VMEM((K, WCOL), jnp.float32),
        pltpu.VMEM((ROWS_PER_TILE, WCOL), jnp.float32),
        pltpu.VMEM_SHARED((BATCH, WCOL), jnp.float32),
        pltpu.SemaphoreType.DMA,
    ],
)(_sc_body)


def _combine_body(p_ref, o_ref):
    s = p_ref[0] + p_ref[1]
    denom = jnp.maximum(s[:, DIM:DIM + 1], 1e-12)
    o_ref[...] = s[:, :DIM] / denom


_combine = pl.pallas_call(
    _combine_body,
    out_shape=jax.ShapeDtypeStruct((BATCH, DIM), jnp.float32),
)


@jax.jit
def kernel(sp_ids_values, sp_ids_segment_ids, sp_weights_values, table):
    ids = sp_ids_values.reshape(NW, CHUNK)
    segs = sp_ids_segment_ids.reshape(NW, NSUB, K)
    w = sp_weights_values.reshape(NW, CHUNK)
    partials = _sc_lookup(ids, segs, w, table)
    return _combine(partials)


# speedup vs baseline: 7.3385x; 1.3011x over previous
import functools

import jax
import jax.numpy as jnp
from jax import lax
from jax.experimental import pallas as pl
from jax.experimental.pallas import tpu as pltpu
from jax.experimental.pallas import tpu_sc as plsc

NNZ = 204800
BATCH = 4096
DIM = 64
NC = 2
NS = 16
NW = NC * NS
CHUNK = NNZ // NW
K = 128
NSUB = CHUNK // K
WCOL = 80
ROWS_PER_TILE = BATCH // NS


def _sc_body(ids_hbm, segs_hbm, w_hbm, table_hbm, out_hbm,
             ids_v, segs_v, w_v, rows_v, rows_v2, sbuf, sbuf2, bounce, accum,
             sem, gsem2, ssem, ssem2):
    cid = lax.axis_index("c")
    sid = lax.axis_index("s")
    wid = cid * NS + sid

    pltpu.sync_copy(ids_hbm.at[wid], ids_v)
    pltpu.sync_copy(segs_hbm.at[wid], segs_v)
    pltpu.sync_copy(w_hbm.at[wid], w_v)

    zeros = jnp.zeros((16,), jnp.float32)

    @pl.loop(0, K)
    def _zero_row(r):
        for c in range(WCOL // 16):
            sbuf[r, pl.ds(c * 16, 16)] = zeros

    pltpu.sync_copy(sbuf, accum.at[pl.ds(sid * ROWS_PER_TILE, K)])
    pltpu.sync_copy(sbuf, accum.at[pl.ds(sid * ROWS_PER_TILE + K, K)])
    plsc.subcore_barrier()

    rows_b = (rows_v, rows_v2)
    sbuf_b = (sbuf, sbuf2)
    gsem_b = (sem, gsem2)
    ssem_b = (ssem, ssem2)

    def _compute(g, rows, dst):
        @pl.loop(0, K // 16)
        def _rows16(rr):
            w16 = w_v[pl.ds(g * K + rr * 16, 16)]
            for r in range(16):
                w = w16[r]
                row = rr * 16 + r
                for c in range(DIM // 16):
                    dst[row, pl.ds(c * 16, 16)] = rows[row, pl.ds(c * 16, 16)] * w
                dst[row, pl.ds(DIM, 16)] = jnp.full((16,), w, jnp.float32)

    pltpu.async_copy(table_hbm.at[ids_v.at[pl.ds(0, K)]], rows_b[0], gsem_b[0])
    pltpu.async_copy(table_hbm.at[ids_v.at[pl.ds(K, K)]], rows_b[1], gsem_b[1])

    @pl.loop(0, NSUB, step=2)
    def _g2(g):
        for b in range(2):
            gb = g + b
            nxt = gb + 2
            pltpu.make_async_copy(table_hbm.at[ids_v.at[pl.ds(gb * K, K)]],
                                  rows_b[b], gsem_b[b]).wait()

            @pl.when(gb >= 2)
            def _():
                pltpu.make_async_copy(sbuf_b[b], accum.at[segs_v.at[gb - 2]],
                                      ssem_b[b]).wait()

            _compute(gb, rows_b[b], sbuf_b[b])

            @pl.when(nxt < NSUB)
            def _():
                pltpu.async_copy(table_hbm.at[ids_v.at[pl.ds(nxt * K, K)]],
                                 rows_b[b], gsem_b[b])

            pltpu.async_copy(sbuf_b[b], accum.at[segs_v.at[gb]],
                             ssem_b[b], add=True)

    pltpu.make_async_copy(sbuf_b[0], accum.at[segs_v.at[NSUB - 2]], ssem_b[0]).wait()
    pltpu.make_async_copy(sbuf_b[1], accum.at[segs_v.at[NSUB - 1]], ssem_b[1]).wait()

    plsc.subcore_barrier()

    pltpu.sync_copy(accum.at[pl.ds(sid * ROWS_PER_TILE, ROWS_PER_TILE)], bounce)
    pltpu.sync_copy(bounce, out_hbm.at[cid, pl.ds(sid * ROWS_PER_TILE, ROWS_PER_TILE)])


_sc_lookup = functools.partial(
    pl.kernel,
    out_type=jax.ShapeDtypeStruct((NC, BATCH, WCOL), jnp.float32),
    mesh=plsc.VectorSubcoreMesh(
        core_axis_name="c", subcore_axis_name="s", num_cores=NC, num_subcores=NS
    ),
    compiler_params=pltpu.CompilerParams(use_tc_tiling_on_sc=False),
    scratch_types=[
        pltpu.VMEM((CHUNK,), jnp.int32),
        pltpu.VMEM((NSUB, K), jnp.int32),
        pltpu.VMEM((CHUNK,), jnp.float32),
        pltpu.VMEM((K, DIM), jnp.float32),
        pltpu.VMEM((K, DIM), jnp.float32),
        pltpu.VMEM((K, WCOL), jnp.float32),
        pltpu.VMEM((K, WCOL), jnp.float32),
        pltpu.VMEM((ROWS_PER_TILE, WCOL), jnp.float32),
        pltpu.VMEM_SHARED((BATCH, WCOL), jnp.float32),
        pltpu.SemaphoreType.DMA,
        pltpu.SemaphoreType.DMA,
        pltpu.SemaphoreType.DMA,
        pltpu.SemaphoreType.DMA,
    ],
)(_sc_body)


def _combine_body(p_ref, o_ref):
    s = p_ref[0] + p_ref[1]
    denom = jnp.maximum(s[:, DIM:DIM + 1], 1e-12)
    o_ref[...] = s[:, :DIM] / denom


_combine = pl.pallas_call(
    _combine_body,
    out_shape=jax.ShapeDtypeStruct((BATCH, DIM), jnp.float32),
)


@jax.jit
def kernel(sp_ids_values, sp_ids_segment_ids, sp_weights_values, table):
    ids = sp_ids_values.reshape(NW, CHUNK)
    segs = sp_ids_segment_ids.reshape(NW, NSUB, K)
    w = sp_weights_values.reshape(NW, CHUNK)
    partials = _sc_lookup(ids, segs, w, table)
    return _combine(partials)
